# Initial kernel scaffold; baseline (speedup 1.0000x reference)
import jax
import jax.numpy as jnp
from jax.experimental import pallas as pl


def kernel(x, edge_index, batch, emb_W, emb_b, Wrel, brel, Wroot, W1, b1, W2, b2):
    raise NotImplementedError("write your pallas kernel here")


# speedup vs baseline: 7.3420x; 7.3420x over previous
import functools

import jax
import jax.numpy as jnp
from jax import lax
from jax.experimental import pallas as pl
from jax.experimental.pallas import tpu as pltpu
from jax.experimental.pallas import tpu_sc as plsc

N = 10000
E = 320000
D = 128
H = 128
G = 64
L = 4

NW = 32
CHUNK = 128
CHUNKS = 80
EPW = CHUNK * CHUNKS
E_PAD = NW * EPW
ACC_R = 10240
ROWS_PER_TILE_ZERO = 640
ROWS_PER_TILE_OUT = 624

_BLK = 2000


@functools.cache
def _build_segsum_sc():
    mesh = plsc.VectorSubcoreMesh(core_axis_name="c", subcore_axis_name="s")
    return functools.partial(
        pl.kernel,
        out_type=jax.ShapeDtypeStruct((2, N, H), jnp.float32),
        mesh=mesh,
        scratch_types=[
            pltpu.VMEM((CHUNKS, CHUNK), jnp.int32),
            pltpu.VMEM((CHUNKS, CHUNK), jnp.int32),
            pltpu.VMEM((CHUNK, H), jnp.float32),
            pltpu.VMEM((16, H), jnp.float32),
            pltpu.VMEM_SHARED((ACC_R, H), jnp.float32),
            pltpu.SemaphoreType.DMA,
        ],
    )(_segsum_sc_body)


def _segsum_sc(A, src_p, dst_p):
    return _build_segsum_sc()(A, src_p, dst_p)


def _segsum_sc_body(a_hbm, srci_hbm, dsti_hbm, out_hbm, srcv, dstv, rows, zbuf, acc, sem):
    cid = lax.axis_index("c")
    sid = lax.axis_index("s")
    wid = sid * 2 + cid

    zero16 = jnp.zeros((16,), jnp.float32)
    for i in range(16):
        for j in range(H // 16):
            zbuf[i, pl.ds(j * 16, 16)] = zero16
    zbase = pl.multiple_of(sid * ROWS_PER_TILE_ZERO, 16)
    for k in range(ROWS_PER_TILE_ZERO // 16):
        pltpu.sync_copy(zbuf, acc.at[pl.ds(zbase + k * 16, 16), :])

    pltpu.sync_copy(srci_hbm.at[pl.ds(wid * CHUNKS, CHUNKS), :], srcv)
    pltpu.sync_copy(dsti_hbm.at[pl.ds(wid * CHUNKS, CHUNKS), :], dstv)
    plsc.subcore_barrier()

    def body(k, carry):
        pltpu.async_copy(a_hbm.at[srcv.at[k]], rows, sem).wait()
        pltpu.sync_copy(rows, acc.at[dstv.at[k]], add=True)
        return carry

    lax.fori_loop(0, CHUNKS, body, 0)
    plsc.subcore_barrier()

    obase = pl.multiple_of(sid * ROWS_PER_TILE_OUT, 8)

    @pl.when(sid < 15)
    def _():
        pltpu.sync_copy(
            acc.at[pl.ds(obase, ROWS_PER_TILE_OUT), :],
            out_hbm.at[cid, pl.ds(obase, ROWS_PER_TILE_OUT), :],
        )

    @pl.when(sid == 15)
    def _():
        last = 15 * ROWS_PER_TILE_OUT
        pltpu.sync_copy(
            acc.at[pl.ds(last, N - last), :],
            out_hbm.at[cid, pl.ds(last, N - last), :],
        )


def _pre_body(x_ref, embw_ref, embb_ref, wrel_ref, wroot_ref, brel_ref, a_ref, b_ref):
    h = jnp.dot(x_ref[...], embw_ref[...], preferred_element_type=jnp.float32)
    h = h + embb_ref[...]
    a_ref[...] = jnp.dot(h, wrel_ref[...], preferred_element_type=jnp.float32)
    b_ref[...] = (
        jnp.dot(h, wroot_ref[...], preferred_element_type=jnp.float32) + brel_ref[...]
    )


def _pre(x, embw, embb, wrel, wroot, brel):
    return pl.pallas_call(
        _pre_body,
        grid=(N // _BLK,),
        in_specs=[
            pl.BlockSpec((_BLK, D), lambda i: (i, 0)),
            pl.BlockSpec((D, H), lambda i: (0, 0)),
            pl.BlockSpec((1, H), lambda i: (0, 0)),
            pl.BlockSpec((H, H), lambda i: (0, 0)),
            pl.BlockSpec((H, H), lambda i: (0, 0)),
            pl.BlockSpec((1, H), lambda i: (0, 0)),
        ],
        out_specs=[
            pl.BlockSpec((_BLK, H), lambda i: (i, 0)),
            pl.BlockSpec((_BLK, H), lambda i: (i, 0)),
        ],
        out_shape=[jax.ShapeDtypeStruct((N, H), jnp.float32)] * 2,
    )(x, embw, embb, wrel, wroot, brel)


def _step_body(s0_ref, s1_ref, bp_ref, wrel_ref, wroot_ref, brel_ref, a_ref, b_ref):
    h = jnp.maximum(s0_ref[...] + s1_ref[...] + bp_ref[...], 0.0)
    a_ref[...] = jnp.dot(h, wrel_ref[...], preferred_element_type=jnp.float32)
    b_ref[...] = (
        jnp.dot(h, wroot_ref[...], preferred_element_type=jnp.float32) + brel_ref[...]
    )


def _step(s0, s1, bp, wrel, wroot, brel):
    return pl.pallas_call(
        _step_body,
        grid=(N // _BLK,),
        in_specs=[
            pl.BlockSpec((_BLK, H), lambda i: (i, 0)),
            pl.BlockSpec((_BLK, H), lambda i: (i, 0)),
            pl.BlockSpec((_BLK, H), lambda i: (i, 0)),
            pl.BlockSpec((H, H), lambda i: (0, 0)),
            pl.BlockSpec((H, H), lambda i: (0, 0)),
            pl.BlockSpec((1, H), lambda i: (0, 0)),
        ],
        out_specs=[
            pl.BlockSpec((_BLK, H), lambda i: (i, 0)),
            pl.BlockSpec((_BLK, H), lambda i: (i, 0)),
        ],
        out_shape=[jax.ShapeDtypeStruct((N, H), jnp.float32)] * 2,
    )(s0, s1, bp, wrel, wroot, brel)


def _final_body(
    s0_ref, s1_ref, bp_ref, batch_ref, w1_ref, b1_ref, w2_ref, b2_ref,
    out_ref, pool_scr, cnt_scr,
):
    i = pl.program_id(0)

    @pl.when(i == 0)
    def _():
        pool_scr[...] = jnp.zeros_like(pool_scr)
        cnt_scr[...] = jnp.zeros_like(cnt_scr)

    h = jnp.maximum(s0_ref[...] + s1_ref[...] + bp_ref[...], 0.0)
    ids = batch_ref[0]
    iota = lax.broadcasted_iota(jnp.int32, (G, _BLK), 0)
    mask_t = (iota == ids).astype(jnp.float32)
    pool_scr[...] += lax.dot_general(
        mask_t, h, (((1,), (0,)), ((), ())), preferred_element_type=jnp.float32
    )
    cnt_scr[...] += lax.dot_general(
        mask_t,
        jnp.ones((_BLK, H), jnp.float32),
        (((1,), (0,)), ((), ())),
        preferred_element_type=jnp.float32,
    )

    @pl.when(i == N // _BLK - 1)
    def _():
        pooled = pool_scr[...] / jnp.maximum(cnt_scr[...], 1.0)
        p1 = jnp.maximum(
            jnp.dot(pooled, w1_ref[...], preferred_element_type=jnp.float32)
            + b1_ref[...],
            0.0,
        )
        out_ref[...] = (
            jnp.dot(p1, w2_ref[...], preferred_element_type=jnp.float32) + b2_ref[...]
        )


def _final(s0, s1, bp, batch3, w1, b1, w2p, b2p):
    return pl.pallas_call(
        _final_body,
        grid=(N // _BLK,),
        in_specs=[
            pl.BlockSpec((_BLK, H), lambda i: (i, 0)),
            pl.BlockSpec((_BLK, H), lambda i: (i, 0)),
            pl.BlockSpec((_BLK, H), lambda i: (i, 0)),
            pl.BlockSpec((1, 1, _BLK), lambda i: (i, 0, 0)),
            pl.BlockSpec((H, H), lambda i: (0, 0)),
            pl.BlockSpec((1, H), lambda i: (0, 0)),
            pl.BlockSpec((H, H), lambda i: (0, 0)),
            pl.BlockSpec((1, H), lambda i: (0, 0)),
        ],
        out_specs=pl.BlockSpec((G, H), lambda i: (0, 0)),
        out_shape=jax.ShapeDtypeStruct((G, H), jnp.float32),
        scratch_shapes=[
            pltpu.VMEM((G, H), jnp.float32),
            pltpu.VMEM((G, H), jnp.float32),
        ],
    )(s0, s1, bp, batch3, w1, b1, w2p, b2p)


def kernel(x, edge_index, batch, emb_W, emb_b, Wrel, brel, Wroot, W1, b1, W2, b2):
    src = edge_index[0]
    dst = edge_index[1]

    pad = E_PAD - E
    pad_ar = jnp.arange(pad, dtype=jnp.int32)
    pad_src = (pad_ar * 97) % N
    pad_dst = N + (pad_ar % 128)
    src_p = jnp.concatenate([src, pad_src]).reshape(NW * CHUNKS, CHUNK)
    dst_p = jnp.concatenate([dst, pad_dst]).reshape(NW * CHUNKS, CHUNK)

    embb = emb_b.reshape(1, H)
    batch3 = batch.reshape(N // _BLK, 1, _BLK)
    b1r = b1.reshape(1, H)
    w2p = jnp.pad(W2, ((0, 0), (0, H - 1)))
    b2p = jnp.pad(b2, (0, H - 1)).reshape(1, H)

    A, B = _pre(x, emb_W, embb, Wrel[0], Wroot[0], brel[0].reshape(1, H))
    for l in range(L):
        S = _segsum_sc(A, src_p, dst_p)
        if l < L - 1:
            A, B = _step(
                S[0], S[1], B, Wrel[l + 1], Wroot[l + 1], brel[l + 1].reshape(1, H)
            )
        else:
            out2 = _final(S[0], S[1], B, batch3, W1, b1r, w2p, b2p)
    return out2[:, 0]


# speedup vs baseline: 9.4656x; 1.2892x over previous
import functools

import jax
import jax.numpy as jnp
from jax import lax
from jax.experimental import pallas as pl
from jax.experimental.pallas import tpu as pltpu
from jax.experimental.pallas import tpu_sc as plsc

N = 10000
E = 320000
D = 128
H = 128
G = 64
L = 4

NW = 32
CHUNK = 128
CHUNKS = 80
EPW = CHUNK * CHUNKS
E_PAD = NW * EPW
ACC_R = 10240
ROWS_PER_TILE_ZERO = 640
ROWS_PER_TILE_OUT = 624

_BLK = 2000


@functools.cache
def _build_segsum_sc():
    mesh = plsc.VectorSubcoreMesh(core_axis_name="c", subcore_axis_name="s")
    return functools.partial(
        pl.kernel,
        out_type=jax.ShapeDtypeStruct((2, N, H), jnp.float32),
        mesh=mesh,
        scratch_types=[
            pltpu.VMEM((CHUNKS, CHUNK), jnp.int32),
            pltpu.VMEM((CHUNK,), jnp.int32),
            pltpu.VMEM((CHUNK,), jnp.int32),
            pltpu.VMEM((CHUNK,), jnp.int32),
            pltpu.VMEM((CHUNK,), jnp.int32),
            pltpu.VMEM((CHUNK, H), jnp.float32),
            pltpu.VMEM((CHUNK, H), jnp.float32),
            pltpu.VMEM((8, H), jnp.float32),
            pltpu.VMEM_SHARED((ACC_R, H), jnp.float32),
            pltpu.SemaphoreType.DMA,
        ],
    )(_segsum_sc_body)


def _segsum_sc(A, packed):
    return _build_segsum_sc()(A, packed)


def _segsum_sc_body(
    a_hbm, pck_hbm, out_hbm, pckv, s_a, s_b, d_a, d_b, rows0, rows1, zbuf, acc, sem
):
    cid = lax.axis_index("c")
    sid = lax.axis_index("s")
    wid = sid * 2 + cid

    zero16 = jnp.zeros((16,), jnp.float32)
    for i in range(8):
        for j in range(H // 16):
            zbuf[i, pl.ds(j * 16, 16)] = zero16
    zbase = pl.multiple_of(sid * ROWS_PER_TILE_ZERO, 8)

    def zbody(k, carry):
        pltpu.sync_copy(zbuf, acc.at[pl.ds(zbase + k * 8, 8), :])
        return carry

    lax.fori_loop(0, ROWS_PER_TILE_ZERO // 8, zbody, 0)

    pltpu.sync_copy(pck_hbm.at[pl.ds(wid * CHUNKS, CHUNKS), :], pckv)
    plsc.subcore_barrier()

    def unpack(k, s_ref, d_ref):
        for i in range(CHUNK // 16):
            pv = pckv[k, pl.ds(i * 16, 16)]
            s_ref[pl.ds(i * 16, 16)] = jnp.bitwise_and(pv, 16383)
            d_ref[pl.ds(i * 16, 16)] = lax.shift_right_logical(pv, 14)

    unpack(0, s_a, d_a)
    pltpu.async_copy(a_hbm.at[s_a], rows0, sem)

    def body(p, carry):
        k0 = p * 2
        unpack(k0 + 1, s_b, d_b)
        pltpu.make_async_copy(a_hbm.at[s_a], rows0, sem).wait()
        pltpu.async_copy(a_hbm.at[s_b], rows1, sem)
        pltpu.sync_copy(rows0, acc.at[d_a], add=True)

        @pl.when(p < CHUNKS // 2 - 1)
        def _():
            unpack(k0 + 2, s_a, d_a)

        pltpu.make_async_copy(a_hbm.at[s_b], rows1, sem).wait()

        @pl.when(p < CHUNKS // 2 - 1)
        def _():
            pltpu.async_copy(a_hbm.at[s_a], rows0, sem)

        pltpu.sync_copy(rows1, acc.at[d_b], add=True)
        return carry

    lax.fori_loop(0, CHUNKS // 2, body, 0)
    plsc.subcore_barrier()

    obase = pl.multiple_of(sid * ROWS_PER_TILE_OUT, 8)

    @pl.when(sid < 15)
    def _():
        pltpu.sync_copy(
            acc.at[pl.ds(obase, ROWS_PER_TILE_OUT), :],
            out_hbm.at[cid, pl.ds(obase, ROWS_PER_TILE_OUT), :],
        )

    @pl.when(sid == 15)
    def _():
        last = 15 * ROWS_PER_TILE_OUT
        pltpu.sync_copy(
            acc.at[pl.ds(last, N - last), :],
            out_hbm.at[cid, pl.ds(last, N - last), :],
        )


def _pre_body(x_ref, embw_ref, embb_ref, wrel_ref, wroot_ref, brel_ref, a_ref, b_ref):
    h = jnp.dot(x_ref[...], embw_ref[...], preferred_element_type=jnp.float32)
    h = h + embb_ref[...]
    a_ref[...] = jnp.dot(h, wrel_ref[...], preferred_element_type=jnp.float32)
    b_ref[...] = (
        jnp.dot(h, wroot_ref[...], preferred_element_type=jnp.float32) + brel_ref[...]
    )


def _pre(x, embw, embb, wrel, wroot, brel):
    return pl.pallas_call(
        _pre_body,
        grid=(N // _BLK,),
        in_specs=[
            pl.BlockSpec((_BLK, D), lambda i: (i, 0)),
            pl.BlockSpec((D, H), lambda i: (0, 0)),
            pl.BlockSpec((1, H), lambda i: (0, 0)),
            pl.BlockSpec((H, H), lambda i: (0, 0)),
            pl.BlockSpec((H, H), lambda i: (0, 0)),
            pl.BlockSpec((1, H), lambda i: (0, 0)),
        ],
        out_specs=[
            pl.BlockSpec((_BLK, H), lambda i: (i, 0)),
            pl.BlockSpec((_BLK, H), lambda i: (i, 0)),
        ],
        out_shape=[jax.ShapeDtypeStruct((N, H), jnp.float32)] * 2,
    )(x, embw, embb, wrel, wroot, brel)


def _step_body(s0_ref, s1_ref, bp_ref, wrel_ref, wroot_ref, brel_ref, a_ref, b_ref):
    h = jnp.maximum(s0_ref[...] + s1_ref[...] + bp_ref[...], 0.0)
    a_ref[...] = jnp.dot(h, wrel_ref[...], preferred_element_type=jnp.float32)
    b_ref[...] = (
        jnp.dot(h, wroot_ref[...], preferred_element_type=jnp.float32) + brel_ref[...]
    )


def _step(s0, s1, bp, wrel, wroot, brel):
    return pl.pallas_call(
        _step_body,
        grid=(N // _BLK,),
        in_specs=[
            pl.BlockSpec((_BLK, H), lambda i: (i, 0)),
            pl.BlockSpec((_BLK, H), lambda i: (i, 0)),
            pl.BlockSpec((_BLK, H), lambda i: (i, 0)),
            pl.BlockSpec((H, H), lambda i: (0, 0)),
            pl.BlockSpec((H, H), lambda i: (0, 0)),
            pl.BlockSpec((1, H), lambda i: (0, 0)),
        ],
        out_specs=[
            pl.BlockSpec((_BLK, H), lambda i: (i, 0)),
            pl.BlockSpec((_BLK, H), lambda i: (i, 0)),
        ],
        out_shape=[jax.ShapeDtypeStruct((N, H), jnp.float32)] * 2,
    )(s0, s1, bp, wrel, wroot, brel)


def _final_body(
    s0_ref, s1_ref, bp_ref, batch_ref, w1_ref, b1_ref, w2_ref, b2_ref,
    out_ref, pool_scr, cnt_scr,
):
    i = pl.program_id(0)

    @pl.when(i == 0)
    def _():
        pool_scr[...] = jnp.zeros_like(pool_scr)
        cnt_scr[...] = jnp.zeros_like(cnt_scr)

    h = jnp.maximum(s0_ref[...] + s1_ref[...] + bp_ref[...], 0.0)
    ids = batch_ref[0]
    iota = lax.broadcasted_iota(jnp.int32, (G, _BLK), 0)
    mask_t = (iota == ids).astype(jnp.float32)
    pool_scr[...] += lax.dot_general(
        mask_t, h, (((1,), (0,)), ((), ())), preferred_element_type=jnp.float32
    )
    cnt_scr[...] += lax.dot_general(
        mask_t,
        jnp.ones((_BLK, H), jnp.float32),
        (((1,), (0,)), ((), ())),
        preferred_element_type=jnp.float32,
    )

    @pl.when(i == N // _BLK - 1)
    def _():
        pooled = pool_scr[...] / jnp.maximum(cnt_scr[...], 1.0)
        p1 = jnp.maximum(
            jnp.dot(pooled, w1_ref[...], preferred_element_type=jnp.float32)
            + b1_ref[...],
            0.0,
        )
        out_ref[...] = (
            jnp.dot(p1, w2_ref[...], preferred_element_type=jnp.float32) + b2_ref[...]
        )


def _final(s0, s1, bp, batch3, w1, b1, w2p, b2p):
    return pl.pallas_call(
        _final_body,
        grid=(N // _BLK,),
        in_specs=[
            pl.BlockSpec((_BLK, H), lambda i: (i, 0)),
            pl.BlockSpec((_BLK, H), lambda i: (i, 0)),
            pl.BlockSpec((_BLK, H), lambda i: (i, 0)),
            pl.BlockSpec((1, 1, _BLK), lambda i: (i, 0, 0)),
            pl.BlockSpec((H, H), lambda i: (0, 0)),
            pl.BlockSpec((1, H), lambda i: (0, 0)),
            pl.BlockSpec((H, H), lambda i: (0, 0)),
            pl.BlockSpec((1, H), lambda i: (0, 0)),
        ],
        out_specs=pl.BlockSpec((G, H), lambda i: (0, 0)),
        out_shape=jax.ShapeDtypeStruct((G, H), jnp.float32),
        scratch_shapes=[
            pltpu.VMEM((G, H), jnp.float32),
            pltpu.VMEM((G, H), jnp.float32),
        ],
    )(s0, s1, bp, batch3, w1, b1, w2p, b2p)


def kernel(x, edge_index, batch, emb_W, emb_b, Wrel, brel, Wroot, W1, b1, W2, b2):
    src = edge_index[0]
    dst = edge_index[1]

    pad = E_PAD - E
    pad_ar = jnp.arange(pad, dtype=jnp.int32)
    pad_src = (pad_ar * 97) % N
    pad_dst = N + (pad_ar % 128)
    src_f = jnp.concatenate([src, pad_src])
    dst_f = jnp.concatenate([dst, pad_dst])
    packed = (dst_f * 16384 + src_f).reshape(NW * CHUNKS, CHUNK)

    embb = emb_b.reshape(1, H)
    batch3 = batch.reshape(N // _BLK, 1, _BLK)
    b1r = b1.reshape(1, H)
    w2p = jnp.pad(W2, ((0, 0), (0, H - 1)))
    b2p = jnp.pad(b2, (0, H - 1)).reshape(1, H)

    A, B = _pre(x, emb_W, embb, Wrel[0], Wroot[0], brel[0].reshape(1, H))
    for l in range(L):
        S = _segsum_sc(A, packed)
        if l < L - 1:
            A, B = _step(
                S[0], S[1], B, Wrel[l + 1], Wroot[l + 1], brel[l + 1].reshape(1, H)
            )
        else:
            out2 = _final(S[0], S[1], B, batch3, W1, b1r, w2p, b2p)
    return out2[:, 0]


# speedup vs baseline: 11.0757x; 1.1701x over previous
import functools

import jax
import jax.numpy as jnp
from jax import lax
from jax.experimental import pallas as pl
from jax.experimental.pallas import tpu as pltpu
from jax.experimental.pallas import tpu_sc as plsc

N = 10000
E = 320000
D = 128
H = 128
G = 64
L = 4

NW = 32
CHUNK = 128
CHUNKS = 80
EPW = CHUNK * CHUNKS
E_PAD = NW * EPW
ACC_R = 10240
ROWS_PER_TILE_ZERO = 640
ROWS_PER_TILE_OUT = 624

_BLK = 2000


@functools.cache
def _build_segsum_sc():
    mesh = plsc.VectorSubcoreMesh(core_axis_name="c", subcore_axis_name="s")
    return functools.partial(
        pl.kernel,
        out_type=jax.ShapeDtypeStruct((2, N, H), jnp.float32),
        mesh=mesh,
        scratch_types=[
            pltpu.VMEM((CHUNKS, CHUNK), jnp.int32),
            pltpu.VMEM((CHUNK,), jnp.int32),
            pltpu.VMEM((CHUNK,), jnp.int32),
            pltpu.VMEM((CHUNK,), jnp.int32),
            pltpu.VMEM((CHUNK,), jnp.int32),
            pltpu.VMEM((CHUNK, H), jnp.float32),
            pltpu.VMEM((CHUNK, H), jnp.float32),
            pltpu.VMEM((8, H), jnp.float32),
            pltpu.VMEM_SHARED((ACC_R, H), jnp.float32),
            pltpu.SemaphoreType.DMA,
        ],
    )(_segsum_sc_body)


def _segsum_sc(A, packed):
    return _build_segsum_sc()(A, packed)


def _segsum_sc_body(
    a_hbm, pck_hbm, out_hbm, pckv, s_a, s_b, d_a, d_b, rows0, rows1, zbuf, acc, sem
):
    cid = lax.axis_index("c")
    sid = lax.axis_index("s")
    wid = sid * 2 + cid

    zero16 = jnp.zeros((16,), jnp.float32)
    for i in range(8):
        for j in range(H // 16):
            zbuf[i, pl.ds(j * 16, 16)] = zero16
    zbase = pl.multiple_of(sid * ROWS_PER_TILE_ZERO, 8)

    def zbody(k, carry):
        pltpu.sync_copy(zbuf, acc.at[pl.ds(zbase + k * 8, 8), :])
        return carry

    lax.fori_loop(0, ROWS_PER_TILE_ZERO // 8, zbody, 0)

    pltpu.sync_copy(pck_hbm.at[pl.ds(wid * CHUNKS, CHUNKS), :], pckv)
    plsc.subcore_barrier()

    def unpack(k, s_ref, d_ref):
        for i in range(CHUNK // 16):
            pv = pckv[k, pl.ds(i * 16, 16)]
            s_ref[pl.ds(i * 16, 16)] = jnp.bitwise_and(pv, 16383)
            d_ref[pl.ds(i * 16, 16)] = lax.shift_right_logical(pv, 14)

    unpack(0, s_a, d_a)
    unpack(1, s_b, d_b)
    pltpu.async_copy(a_hbm.at[s_a], rows0, sem)
    pltpu.async_copy(a_hbm.at[s_b], rows1, sem)

    def body(p, carry):
        k0 = p * 2
        pltpu.make_async_copy(a_hbm.at[s_a], rows0, sem).wait()
        pltpu.sync_copy(rows0, acc.at[d_a], add=True)

        @pl.when(p < CHUNKS // 2 - 1)
        def _():
            unpack(k0 + 2, s_a, d_a)
            pltpu.async_copy(a_hbm.at[s_a], rows0, sem)

        pltpu.make_async_copy(a_hbm.at[s_b], rows1, sem).wait()
        pltpu.sync_copy(rows1, acc.at[d_b], add=True)

        @pl.when(p < CHUNKS // 2 - 1)
        def _():
            unpack(k0 + 3, s_b, d_b)
            pltpu.async_copy(a_hbm.at[s_b], rows1, sem)

        return carry

    lax.fori_loop(0, CHUNKS // 2, body, 0)
    plsc.subcore_barrier()

    obase = pl.multiple_of(sid * ROWS_PER_TILE_OUT, 8)

    @pl.when(sid < 15)
    def _():
        pltpu.sync_copy(
            acc.at[pl.ds(obase, ROWS_PER_TILE_OUT), :],
            out_hbm.at[cid, pl.ds(obase, ROWS_PER_TILE_OUT), :],
        )

    @pl.when(sid == 15)
    def _():
        last = 15 * ROWS_PER_TILE_OUT
        pltpu.sync_copy(
            acc.at[pl.ds(last, N - last), :],
            out_hbm.at[cid, pl.ds(last, N - last), :],
        )


def _pre_body(x_ref, embw_ref, embb_ref, wrel_ref, wroot_ref, brel_ref, a_ref, b_ref):
    h = jnp.dot(x_ref[...], embw_ref[...], preferred_element_type=jnp.float32)
    h = h + embb_ref[...]
    a_ref[...] = jnp.dot(h, wrel_ref[...], preferred_element_type=jnp.float32)
    b_ref[...] = (
        jnp.dot(h, wroot_ref[...], preferred_element_type=jnp.float32) + brel_ref[...]
    )


def _pre(x, embw, embb, wrel, wroot, brel):
    return pl.pallas_call(
        _pre_body,
        grid=(N // _BLK,),
        in_specs=[
            pl.BlockSpec((_BLK, D), lambda i: (i, 0)),
            pl.BlockSpec((D, H), lambda i: (0, 0)),
            pl.BlockSpec((1, H), lambda i: (0, 0)),
            pl.BlockSpec((H, H), lambda i: (0, 0)),
            pl.BlockSpec((H, H), lambda i: (0, 0)),
            pl.BlockSpec((1, H), lambda i: (0, 0)),
        ],
        out_specs=[
            pl.BlockSpec((_BLK, H), lambda i: (i, 0)),
            pl.BlockSpec((_BLK, H), lambda i: (i, 0)),
        ],
        out_shape=[jax.ShapeDtypeStruct((N, H), jnp.float32)] * 2,
    )(x, embw, embb, wrel, wroot, brel)


def _step_body(s0_ref, s1_ref, bp_ref, wrel_ref, wroot_ref, brel_ref, a_ref, b_ref):
    h = jnp.maximum(s0_ref[...] + s1_ref[...] + bp_ref[...], 0.0)
    a_ref[...] = jnp.dot(h, wrel_ref[...], preferred_element_type=jnp.float32)
    b_ref[...] = (
        jnp.dot(h, wroot_ref[...], preferred_element_type=jnp.float32) + brel_ref[...]
    )


def _step(s0, s1, bp, wrel, wroot, brel):
    return pl.pallas_call(
        _step_body,
        grid=(N // _BLK,),
        in_specs=[
            pl.BlockSpec((_BLK, H), lambda i: (i, 0)),
            pl.BlockSpec((_BLK, H), lambda i: (i, 0)),
            pl.BlockSpec((_BLK, H), lambda i: (i, 0)),
            pl.BlockSpec((H, H), lambda i: (0, 0)),
            pl.BlockSpec((H, H), lambda i: (0, 0)),
            pl.BlockSpec((1, H), lambda i: (0, 0)),
        ],
        out_specs=[
            pl.BlockSpec((_BLK, H), lambda i: (i, 0)),
            pl.BlockSpec((_BLK, H), lambda i: (i, 0)),
        ],
        out_shape=[jax.ShapeDtypeStruct((N, H), jnp.float32)] * 2,
    )(s0, s1, bp, wrel, wroot, brel)


def _final_body(
    s0_ref, s1_ref, bp_ref, batch_ref, w1_ref, b1_ref, w2_ref, b2_ref,
    out_ref, pool_scr, cnt_scr,
):
    i = pl.program_id(0)

    @pl.when(i == 0)
    def _():
        pool_scr[...] = jnp.zeros_like(pool_scr)
        cnt_scr[...] = jnp.zeros_like(cnt_scr)

    h = jnp.maximum(s0_ref[...] + s1_ref[...] + bp_ref[...], 0.0)
    ids = batch_ref[0]
    iota = lax.broadcasted_iota(jnp.int32, (G, _BLK), 0)
    mask_t = (iota == ids).astype(jnp.float32)
    pool_scr[...] += lax.dot_general(
        mask_t, h, (((1,), (0,)), ((), ())), preferred_element_type=jnp.float32
    )
    cnt_scr[...] += lax.dot_general(
        mask_t,
        jnp.ones((_BLK, H), jnp.float32),
        (((1,), (0,)), ((), ())),
        preferred_element_type=jnp.float32,
    )

    @pl.when(i == N // _BLK - 1)
    def _():
        pooled = pool_scr[...] / jnp.maximum(cnt_scr[...], 1.0)
        p1 = jnp.maximum(
            jnp.dot(pooled, w1_ref[...], preferred_element_type=jnp.float32)
            + b1_ref[...],
            0.0,
        )
        out_ref[...] = (
            jnp.dot(p1, w2_ref[...], preferred_element_type=jnp.float32) + b2_ref[...]
        )


def _final(s0, s1, bp, batch3, w1, b1, w2p, b2p):
    return pl.pallas_call(
        _final_body,
        grid=(N // _BLK,),
        in_specs=[
            pl.BlockSpec((_BLK, H), lambda i: (i, 0)),
            pl.BlockSpec((_BLK, H), lambda i: (i, 0)),
            pl.BlockSpec((_BLK, H), lambda i: (i, 0)),
            pl.BlockSpec((1, 1, _BLK), lambda i: (i, 0, 0)),
            pl.BlockSpec((H, H), lambda i: (0, 0)),
            pl.BlockSpec((1, H), lambda i: (0, 0)),
            pl.BlockSpec((H, H), lambda i: (0, 0)),
            pl.BlockSpec((1, H), lambda i: (0, 0)),
        ],
        out_specs=pl.BlockSpec((G, H), lambda i: (0, 0)),
        out_shape=jax.ShapeDtypeStruct((G, H), jnp.float32),
        scratch_shapes=[
            pltpu.VMEM((G, H), jnp.float32),
            pltpu.VMEM((G, H), jnp.float32),
        ],
    )(s0, s1, bp, batch3, w1, b1, w2p, b2p)


def kernel(x, edge_index, batch, emb_W, emb_b, Wrel, brel, Wroot, W1, b1, W2, b2):
    src = edge_index[0]
    dst = edge_index[1]

    pad = E_PAD - E
    pad_ar = jnp.arange(pad, dtype=jnp.int32)
    pad_src = (pad_ar * 97) % N
    pad_dst = N + (pad_ar % 128)
    src_f = jnp.concatenate([src, pad_src])
    dst_f = jnp.concatenate([dst, pad_dst])
    packed = (dst_f * 16384 + src_f).reshape(NW * CHUNKS, CHUNK)

    embb = emb_b.reshape(1, H)
    batch3 = batch.reshape(N // _BLK, 1, _BLK)
    b1r = b1.reshape(1, H)
    w2p = jnp.pad(W2, ((0, 0), (0, H - 1)))
    b2p = jnp.pad(b2, (0, H - 1)).reshape(1, H)

    A, B = _pre(x, emb_W, embb, Wrel[0], Wroot[0], brel[0].reshape(1, H))
    for l in range(L):
        S = _segsum_sc(A, packed)
        if l < L - 1:
            A, B = _step(
                S[0], S[1], B, Wrel[l + 1], Wroot[l + 1], brel[l + 1].reshape(1, H)
            )
        else:
            out2 = _final(S[0], S[1], B, batch3, W1, b1r, w2p, b2p)
    return out2[:, 0]


# speedup vs baseline: 12.3534x; 1.1154x over previous
import functools

import jax
import jax.numpy as jnp
from jax import lax
from jax.experimental import pallas as pl
from jax.experimental.pallas import tpu as pltpu
from jax.experimental.pallas import tpu_sc as plsc

N = 10000
E = 320000
D = 128
H = 128
G = 64
L = 4

NW = 32
CHUNK = 64
CHUNKS = 160
NBUF = 4
EPW = CHUNK * CHUNKS
E_PAD = NW * EPW
ACC_R = 10240
ROWS_PER_TILE_ZERO = 640
ROWS_PER_TILE_OUT = 624

_BLK = 2000


@functools.cache
def _build_segsum_sc():
    mesh = plsc.VectorSubcoreMesh(core_axis_name="c", subcore_axis_name="s")
    return functools.partial(
        pl.kernel,
        out_type=jax.ShapeDtypeStruct((2, N, H), jnp.float32),
        mesh=mesh,
        scratch_types=[
            pltpu.VMEM((EPW,), jnp.int32),
            pltpu.VMEM((NBUF, CHUNK), jnp.int32),
            pltpu.VMEM((NBUF, CHUNK), jnp.int32),
            pltpu.VMEM((NBUF, CHUNK, H), jnp.float32),
            pltpu.VMEM((8, H), jnp.float32),
            pltpu.VMEM_SHARED((ACC_R, H), jnp.float32),
            pltpu.SemaphoreType.DMA,
        ],
    )(_segsum_sc_body)


def _segsum_sc(A, packed):
    return _build_segsum_sc()(A, packed)


def _segsum_sc_body(a_hbm, pck_hbm, out_hbm, pckv, sidx, didx, rows, zbuf, acc, sem):
    cid = lax.axis_index("c")
    sid = lax.axis_index("s")
    wid = sid * 2 + cid

    zero16 = jnp.zeros((16,), jnp.float32)
    for i in range(8):
        for j in range(H // 16):
            zbuf[i, pl.ds(j * 16, 16)] = zero16
    zbase = pl.multiple_of(sid * ROWS_PER_TILE_ZERO, 8)

    def zbody(k, carry):
        pltpu.sync_copy(zbuf, acc.at[pl.ds(zbase + k * 8, 8), :])
        return carry

    lax.fori_loop(0, ROWS_PER_TILE_ZERO // 8, zbody, 0)

    pltpu.sync_copy(pck_hbm.at[pl.ds(wid * EPW, EPW)], pckv)
    plsc.subcore_barrier()

    def unpack(k, b):
        for i in range(CHUNK // 16):
            pv = pckv[pl.ds(k * CHUNK + i * 16, 16)]
            sidx[b, pl.ds(i * 16, 16)] = jnp.bitwise_and(pv, 16383)
            didx[b, pl.ds(i * 16, 16)] = lax.shift_right_logical(pv, 14)

    for b in range(NBUF):
        unpack(b, b)
        pltpu.async_copy(a_hbm.at[sidx.at[b]], rows.at[b], sem)

    def body(p, carry):
        k0 = p * NBUF
        for b in range(NBUF):
            pltpu.make_async_copy(a_hbm.at[sidx.at[b]], rows.at[b], sem).wait()
            pltpu.sync_copy(rows.at[b], acc.at[didx.at[b]], add=True)

            @pl.when(p < CHUNKS // NBUF - 1)
            def _():
                unpack(k0 + NBUF + b, b)
                pltpu.async_copy(a_hbm.at[sidx.at[b]], rows.at[b], sem)

        return carry

    lax.fori_loop(0, CHUNKS // NBUF, body, 0)
    plsc.subcore_barrier()

    obase = pl.multiple_of(sid * ROWS_PER_TILE_OUT, 8)

    @pl.when(sid < 15)
    def _():
        pltpu.sync_copy(
            acc.at[pl.ds(obase, ROWS_PER_TILE_OUT), :],
            out_hbm.at[cid, pl.ds(obase, ROWS_PER_TILE_OUT), :],
        )

    @pl.when(sid == 15)
    def _():
        last = 15 * ROWS_PER_TILE_OUT
        pltpu.sync_copy(
            acc.at[pl.ds(last, N - last), :],
            out_hbm.at[cid, pl.ds(last, N - last), :],
        )


def _pre_body(x_ref, embw_ref, embb_ref, wrel_ref, wroot_ref, brel_ref, a_ref, b_ref):
    h = jnp.dot(x_ref[...], embw_ref[...], preferred_element_type=jnp.float32)
    h = h + embb_ref[...]
    a_ref[...] = jnp.dot(h, wrel_ref[...], preferred_element_type=jnp.float32)
    b_ref[...] = (
        jnp.dot(h, wroot_ref[...], preferred_element_type=jnp.float32) + brel_ref[...]
    )


def _pre(x, embw, embb, wrel, wroot, brel):
    return pl.pallas_call(
        _pre_body,
        grid=(N // _BLK,),
        in_specs=[
            pl.BlockSpec((_BLK, D), lambda i: (i, 0)),
            pl.BlockSpec((D, H), lambda i: (0, 0)),
            pl.BlockSpec((1, H), lambda i: (0, 0)),
            pl.BlockSpec((H, H), lambda i: (0, 0)),
            pl.BlockSpec((H, H), lambda i: (0, 0)),
            pl.BlockSpec((1, H), lambda i: (0, 0)),
        ],
        out_specs=[
            pl.BlockSpec((_BLK, H), lambda i: (i, 0)),
            pl.BlockSpec((_BLK, H), lambda i: (i, 0)),
        ],
        out_shape=[jax.ShapeDtypeStruct((N, H), jnp.float32)] * 2,
    )(x, embw, embb, wrel, wroot, brel)


def _step_body(s0_ref, s1_ref, bp_ref, wrel_ref, wroot_ref, brel_ref, a_ref, b_ref):
    h = jnp.maximum(s0_ref[...] + s1_ref[...] + bp_ref[...], 0.0)
    a_ref[...] = jnp.dot(h, wrel_ref[...], preferred_element_type=jnp.float32)
    b_ref[...] = (
        jnp.dot(h, wroot_ref[...], preferred_element_type=jnp.float32) + brel_ref[...]
    )


def _step(s0, s1, bp, wrel, wroot, brel):
    return pl.pallas_call(
        _step_body,
        grid=(N // _BLK,),
        in_specs=[
            pl.BlockSpec((_BLK, H), lambda i: (i, 0)),
            pl.BlockSpec((_BLK, H), lambda i: (i, 0)),
            pl.BlockSpec((_BLK, H), lambda i: (i, 0)),
            pl.BlockSpec((H, H), lambda i: (0, 0)),
            pl.BlockSpec((H, H), lambda i: (0, 0)),
            pl.BlockSpec((1, H), lambda i: (0, 0)),
        ],
        out_specs=[
            pl.BlockSpec((_BLK, H), lambda i: (i, 0)),
            pl.BlockSpec((_BLK, H), lambda i: (i, 0)),
        ],
        out_shape=[jax.ShapeDtypeStruct((N, H), jnp.float32)] * 2,
    )(s0, s1, bp, wrel, wroot, brel)


def _final_body(
    s0_ref, s1_ref, bp_ref, batch_ref, w1_ref, b1_ref, w2_ref, b2_ref,
    out_ref, pool_scr, cnt_scr,
):
    i = pl.program_id(0)

    @pl.when(i == 0)
    def _():
        pool_scr[...] = jnp.zeros_like(pool_scr)
        cnt_scr[...] = jnp.zeros_like(cnt_scr)

    h = jnp.maximum(s0_ref[...] + s1_ref[...] + bp_ref[...], 0.0)
    ids = batch_ref[0]
    iota = lax.broadcasted_iota(jnp.int32, (G, _BLK), 0)
    mask_t = (iota == ids).astype(jnp.float32)
    pool_scr[...] += lax.dot_general(
        mask_t, h, (((1,), (0,)), ((), ())), preferred_element_type=jnp.float32
    )
    cnt_scr[...] += lax.dot_general(
        mask_t,
        jnp.ones((_BLK, H), jnp.float32),
        (((1,), (0,)), ((), ())),
        preferred_element_type=jnp.float32,
    )

    @pl.when(i == N // _BLK - 1)
    def _():
        pooled = pool_scr[...] / jnp.maximum(cnt_scr[...], 1.0)
        p1 = jnp.maximum(
            jnp.dot(pooled, w1_ref[...], preferred_element_type=jnp.float32)
            + b1_ref[...],
            0.0,
        )
        out_ref[...] = (
            jnp.dot(p1, w2_ref[...], preferred_element_type=jnp.float32) + b2_ref[...]
        )


def _final(s0, s1, bp, batch3, w1, b1, w2p, b2p):
    return pl.pallas_call(
        _final_body,
        grid=(N // _BLK,),
        in_specs=[
            pl.BlockSpec((_BLK, H), lambda i: (i, 0)),
            pl.BlockSpec((_BLK, H), lambda i: (i, 0)),
            pl.BlockSpec((_BLK, H), lambda i: (i, 0)),
            pl.BlockSpec((1, 1, _BLK), lambda i: (i, 0, 0)),
            pl.BlockSpec((H, H), lambda i: (0, 0)),
            pl.BlockSpec((1, H), lambda i: (0, 0)),
            pl.BlockSpec((H, H), lambda i: (0, 0)),
            pl.BlockSpec((1, H), lambda i: (0, 0)),
        ],
        out_specs=pl.BlockSpec((G, H), lambda i: (0, 0)),
        out_shape=jax.ShapeDtypeStruct((G, H), jnp.float32),
        scratch_shapes=[
            pltpu.VMEM((G, H), jnp.float32),
            pltpu.VMEM((G, H), jnp.float32),
        ],
    )(s0, s1, bp, batch3, w1, b1, w2p, b2p)


def kernel(x, edge_index, batch, emb_W, emb_b, Wrel, brel, Wroot, W1, b1, W2, b2):
    src = edge_index[0]
    dst = edge_index[1]

    pad = E_PAD - E
    pad_ar = jnp.arange(pad, dtype=jnp.int32)
    pad_src = (pad_ar * 97) % N
    pad_dst = N + (pad_ar % 128)
    src_f = jnp.concatenate([src, pad_src])
    dst_f = jnp.concatenate([dst, pad_dst])
    packed = dst_f * 16384 + src_f

    embb = emb_b.reshape(1, H)
    batch3 = batch.reshape(N // _BLK, 1, _BLK)
    b1r = b1.reshape(1, H)
    w2p = jnp.pad(W2, ((0, 0), (0, H - 1)))
    b2p = jnp.pad(b2, (0, H - 1)).reshape(1, H)

    A, B = _pre(x, emb_W, embb, Wrel[0], Wroot[0], brel[0].reshape(1, H))
    for l in range(L):
        S = _segsum_sc(A, packed)
        if l < L - 1:
            A, B = _step(
                S[0], S[1], B, Wrel[l + 1], Wroot[l + 1], brel[l + 1].reshape(1, H)
            )
        else:
            out2 = _final(S[0], S[1], B, batch3, W1, b1r, w2p, b2p)
    return out2[:, 0]
